# Initial kernel scaffold; baseline (speedup 1.0000x reference)
import jax
import jax.numpy as jnp
from jax.experimental import pallas as pl


def kernel(b1, s1, t1, c2, s2, t2, wf1, bf1, wf2, bf2, wf3, bf3, x_nchw):
    raise NotImplementedError("write your pallas kernel here")


# speedup vs baseline: 1.4046x; 1.4046x over previous
import numpy as np
import jax
import jax.numpy as jnp
from jax.experimental import pallas as pl
from jax.experimental.pallas import tpu as pltpu


def _fused_kernel(x_ref, b1_ref, s1_ref, t1_ref, c2_ref, s2_ref, t2_ref,
                  wf1_ref, bf1_ref, wf2_ref, bf2_ref, wf3_ref, bf3_ref,
                  out_ref):
    rt = x_ref.shape[1]
    tb = rt // 8
    r2 = rt - 2

    def mm(a, b):
        return jnp.dot(a, b, preferred_element_type=jnp.float32)

    def bn_relu_pool(z, s, t):
        z = jnp.maximum(z * s + t, 0.0)
        return jnp.maximum(jnp.maximum(z[:, 0:128], z[:, 128:256]),
                           jnp.maximum(z[:, 256:384], z[:, 384:512]))

    x01 = x_ref[0]
    x23 = x_ref[1]

    z_e = mm(x01, b1_ref[0]) + mm(x23, b1_ref[1])
    z_o = mm(x23[0:rt - 1], b1_ref[0]) + mm(x01[1:rt], b1_ref[1])
    p_e = bn_relu_pool(z_e, s1_ref[...], t1_ref[...]).astype(jnp.bfloat16)
    p_o = bn_relu_pool(z_o, s1_ref[...], t1_ref[...]).astype(jnp.bfloat16)

    pk = jnp.concatenate([p_e[0:r2], p_o[0:r2]], axis=1)
    pk1 = jnp.concatenate([p_e[1:r2 + 1], p_o[1:r2 + 1]], axis=1)
    z2 = mm(pk, c2_ref[0]) + mm(pk1, c2_ref[1])
    p2 = bn_relu_pool(z2, s2_ref[...], t2_ref[...])

    p2p = jnp.pad(p2, ((0, 2), (0, 0)))
    p3 = p2p.reshape(tb, 8, 128)
    acc = bf1_ref[...]
    for po in range(6):
        fpo = p3[:, po, :].astype(jnp.bfloat16)
        acc = acc + mm(fpo, wf1_ref[po])
    h1 = jnp.maximum(acc, 0.0).astype(jnp.bfloat16)

    h2 = jnp.maximum(mm(h1, wf2_ref[...]) + bf2_ref[...], 0.0).astype(jnp.bfloat16)
    out_ref[...] = mm(h2, wf3_ref[...]) + bf3_ref[...]


def _plane_perm():
    idx = np.arange(256)
    for half in (0, 128):
        for c in range(3):
            for w in range(30):
                idx[half + c * 30 + w] = half + w * 3 + c
    return idx


_PERM = _plane_perm()


def _prep_input(x_nchw, n_pad):
    n = x_nchw.shape[0]
    x = x_nchw.astype(jnp.bfloat16)
    x = jnp.transpose(x, (0, 2, 1, 3)).reshape(n, 30, 90)
    x = jnp.pad(x, ((0, 0), (0, 2), (0, 38)))
    x = x.reshape(n, 8, 4, 128)
    x01 = jnp.concatenate([x[:, :, 0, :], x[:, :, 1, :]], axis=-1)
    x23 = jnp.concatenate([x[:, :, 2, :], x[:, :, 3, :]], axis=-1)
    xs = jnp.stack([x01, x23], axis=0).reshape(2, n * 8, 256)
    if n_pad > n:
        xs = jnp.pad(xs, ((0, 0), (0, (n_pad - n) * 8), (0, 0)))
    return xs


@jax.jit
def _forward(b1, s1, t1, c2, s2, t2, wf1, bf1, wf2, bf2, wf3, bf3, x_nchw):
    n = x_nchw.shape[0]
    tb = min(128, max(8, ((n + 1) // 2 + 7) // 8 * 8))
    n_pad = ((n + tb - 1) // tb) * tb
    rt = tb * 8
    xs = _prep_input(x_nchw, n_pad)
    b1p = b1[:, _PERM, :]
    wf1r = wf1.reshape(6, 128, 128)
    grid = (n_pad // tb,)

    c2d = lambda i: (0, 0)
    c3d = lambda i: (0, 0, 0)
    in_specs = [
        pl.BlockSpec((2, rt, 256), lambda i: (0, i, 0)),
        pl.BlockSpec((2, 256, 512), c3d),
        pl.BlockSpec((1, 512), c2d), pl.BlockSpec((1, 512), c2d),
        pl.BlockSpec((2, 256, 512), c3d),
        pl.BlockSpec((1, 512), c2d), pl.BlockSpec((1, 512), c2d),
        pl.BlockSpec((6, 128, 128), c3d),
        pl.BlockSpec((1, 128), c2d),
        pl.BlockSpec((128, 128), c2d), pl.BlockSpec((1, 128), c2d),
        pl.BlockSpec((128, 128), c2d), pl.BlockSpec((1, 128), c2d),
    ]
    out_specs = pl.BlockSpec((tb, 128), lambda i: (i, 0))

    flops = grid[0] * 2 * (2 * rt * 256 * 512 + 2 * (rt - 1) * 256 * 512
                           + 2 * (rt - 2) * 256 * 512
                           + tb * (6 * 128 * 128 + 128 * 128 + 128 * 128))
    bytes_accessed = (xs.size * 2 + n_pad * 128 * 4
                      + (4 * 256 * 512 + 768 * 128 + 2 * 128 * 128) * 2
                      + (4 * 512 + 4 * 128) * 4)

    out = pl.pallas_call(
        _fused_kernel,
        out_shape=jax.ShapeDtypeStruct((n_pad, 128), jnp.float32),
        grid=grid,
        in_specs=in_specs,
        out_specs=out_specs,
        compiler_params=pltpu.CompilerParams(
            dimension_semantics=("parallel",),
            vmem_limit_bytes=64 * 1024 * 1024),
        cost_estimate=pl.CostEstimate(flops=flops, transcendentals=0,
                                      bytes_accessed=bytes_accessed),
    )(xs, b1p, s1, t1, c2, s2, t2, wf1r, bf1, wf2, bf2, wf3, bf3)
    return out[:n, :10]


def kernel(b1, s1, t1, c2, s2, t2, wf1, bf1, wf2, bf2, wf3, bf3, x_nchw):
    return _forward(b1, s1, t1, c2, s2, t2, wf1, bf1, wf2, bf2, wf3, bf3,
                    x_nchw)
